# baseline (speedup 1.0000x reference)
import functools

import jax
import jax.numpy as jnp
from jax.experimental import pallas as pl
from jax.experimental.pallas import tpu as pltpu

_TOPK = 8
_BETA = 1.0
_KB = 4096
_NC = 16
_CW = _KB // _NC
_QB = 128
_BB = 128
_COL_MASK = _CW - 1
_INT_MAX = jnp.iinfo(jnp.int32).max
_PAD_VAL = 1e17


def _knn_body(x_ref, q_ref, idx_ref, out_ref, top_ref, *, nb, nbq):
    i = pl.program_id(0)
    j = pl.program_id(1)
    rows = pl.ds(j * _QB, _QB)

    @pl.when(i == 0)
    def _init():
        top_ref[rows, :] = jnp.full((_QB, _TOPK), jnp.inf, jnp.float32)

    q = q_ref[...]
    xt = x_ref[...]
    g = jax.lax.dot_general(q, xt, (((1,), (0,)), ((), ())),
                            preferred_element_type=jnp.float32)
    q2 = jnp.sum(q * q, axis=1, keepdims=True)
    x2 = jnp.sum(xt * xt, axis=0, keepdims=True)
    d2 = q2 + (x2 - 2.0 * g)

    u = jax.lax.bitcast_convert_type(d2, jnp.int32)
    col = jax.lax.broadcasted_iota(jnp.int32, (_QB, _CW), 1)
    keys = tuple(
        (jax.lax.slice(u, (0, c * _CW), (_QB, (c + 1) * _CW)) & ~_COL_MASK)
        | col
        for c in range(_NC))

    top = top_ref[rows, :]
    neg_inf = jnp.full((_QB, 1), -jnp.inf, jnp.float32)

    def _val(m):
        return jnp.maximum(
            jax.lax.bitcast_convert_type(m & ~_COL_MASK, jnp.float32), 0.0)

    def _mins(keys):
        return tuple(jnp.min(kc, axis=1, keepdims=True) for kc in keys)

    def _cond(carry):
        _, top, rs = carry
        m = functools.reduce(jnp.minimum, rs)
        return jnp.any(_val(m) < top[:, _TOPK - 1:])

    def _body(carry):
        keys, top, rs = carry
        for r in rs:
            v = _val(r)
            shifted = jnp.concatenate([neg_inf, top[:, :_TOPK - 1]], axis=1)
            top = jnp.minimum(jnp.maximum(v, shifted), top)
        keys = tuple(jnp.where(kc == r, _INT_MAX, kc)
                     for kc, r in zip(keys, rs))
        return keys, top, _mins(keys)

    _, top, _ = jax.lax.while_loop(_cond, _body, (keys, top, _mins(keys)))
    top_ref[rows, :] = top

    @pl.when((i == nb - 1) & (j == nbq - 1))
    def _final():
        w = jnp.exp(-_BETA * jnp.sqrt(top_ref[...] + 1e-12))
        nq = w.shape[0]
        nbb = idx_ref.shape[0] // _BB
        for bi in range(nbb):
            brows = pl.ds(bi * _BB, _BB)
            idx = idx_ref[brows, :]
            q_iota = jax.lax.broadcasted_iota(
                jnp.int32, (_BB, nq), 1)
            onehot = (q_iota == idx).astype(jnp.float32)
            out_ref[brows, :] = jax.lax.dot_general(
                onehot, w, (((1,), (0,)), ((), ())),
                preferred_element_type=jnp.float32)


@jax.jit
def kernel(index_data, query_data, indices):
    k, d = index_data.shape
    q, _ = query_data.shape
    b = indices.shape[0]
    nb = pl.cdiv(k, _KB)
    kp = nb * _KB
    nbq = q // _QB
    if kp != k:
        index_data = jnp.pad(index_data, ((0, kp - k), (0, 0)),
                             constant_values=_PAD_VAL)
    xt = index_data.T
    idx2 = indices.reshape(b, 1)

    return pl.pallas_call(
        functools.partial(_knn_body, nb=nb, nbq=nbq),
        grid=(nb, nbq),
        in_specs=[
            pl.BlockSpec((d, _KB), lambda i, j: (0, i)),
            pl.BlockSpec((_QB, d), lambda i, j: (j, 0)),
            pl.BlockSpec((b, 1), lambda i, j: (0, 0)),
        ],
        out_specs=pl.BlockSpec((b, _TOPK), lambda i, j: (0, 0)),
        out_shape=jax.ShapeDtypeStruct((b, _TOPK), jnp.float32),
        scratch_shapes=[pltpu.VMEM((q, _TOPK), jnp.float32)],
    )(xt, query_data, idx2)


# speedup vs baseline: 1.6211x; 1.6211x over previous
import functools

import jax
import jax.numpy as jnp
from jax.experimental import pallas as pl
from jax.experimental.pallas import tpu as pltpu

_TOPK = 8
_BETA = 1.0
_KB = 4096
_NC = 8
_CW = _KB // _NC
_QB = 128
_BB = 128
_COL_MASK = _CW - 1
_INT_MAX = jnp.iinfo(jnp.int32).max
_PAD_VAL = 1e17


def _knn_body(x_ref, q_ref, idx_ref, out_ref, top_ref, key_ref, *, nb, nbq):
    i = pl.program_id(0)
    j = pl.program_id(1)
    rows = pl.ds(j * _QB, _QB)

    @pl.when(i == 0)
    def _init():
        top_ref[rows, :] = jnp.full((_QB, _TOPK), jnp.inf, jnp.float32)

    q = q_ref[...]
    xt = x_ref[...]
    g = jax.lax.dot_general(q, xt, (((1,), (0,)), ((), ())),
                            preferred_element_type=jnp.float32)
    q2 = jnp.sum(q * q, axis=1, keepdims=True)
    x2 = jnp.sum(xt * xt, axis=0, keepdims=True)
    d2 = q2 + (x2 - 2.0 * g)

    u = jax.lax.bitcast_convert_type(d2, jnp.int32)
    col = jax.lax.broadcasted_iota(jnp.int32, (_QB, _CW), 1)
    rs0 = []
    for c in range(_NC):
        kc = (jax.lax.slice(u, (0, c * _CW), (_QB, (c + 1) * _CW))
              & ~_COL_MASK) | col
        key_ref[:, c * _CW:(c + 1) * _CW] = kc
        rs0.append(jnp.min(kc, axis=1, keepdims=True))

    top = top_ref[rows, :]
    neg_inf = jnp.full((_QB, 1), -jnp.inf, jnp.float32)

    def _val(m):
        return jnp.maximum(
            jax.lax.bitcast_convert_type(m & ~_COL_MASK, jnp.float32), 0.0)

    def _cond(carry):
        top, rs = carry
        m = functools.reduce(jnp.minimum, rs)
        return jnp.any(_val(m) < top[:, _TOPK - 1:])

    def _body(carry):
        top, rs = carry
        for r in rs:
            v = _val(r)
            shifted = jnp.concatenate([neg_inf, top[:, :_TOPK - 1]], axis=1)
            top = jnp.minimum(jnp.maximum(v, shifted), top)
        new_rs = []
        for c in range(_NC):
            kc = key_ref[:, c * _CW:(c + 1) * _CW]
            masked = jnp.where(kc <= rs[c], _INT_MAX, kc)
            new_rs.append(jnp.min(masked, axis=1, keepdims=True))
        return top, tuple(new_rs)

    top, _ = jax.lax.while_loop(_cond, _body, (top, tuple(rs0)))
    top_ref[rows, :] = top

    @pl.when((i == nb - 1) & (j == nbq - 1))
    def _final():
        w = jnp.exp(-_BETA * jnp.sqrt(top_ref[...] + 1e-12))
        nq = w.shape[0]
        nbb = idx_ref.shape[0] // _BB
        for bi in range(nbb):
            brows = pl.ds(bi * _BB, _BB)
            idx = idx_ref[brows, :]
            q_iota = jax.lax.broadcasted_iota(
                jnp.int32, (_BB, nq), 1)
            onehot = (q_iota == idx).astype(jnp.float32)
            out_ref[brows, :] = jax.lax.dot_general(
                onehot, w, (((1,), (0,)), ((), ())),
                preferred_element_type=jnp.float32)


@jax.jit
def kernel(index_data, query_data, indices):
    k, d = index_data.shape
    q, _ = query_data.shape
    b = indices.shape[0]
    nb = pl.cdiv(k, _KB)
    kp = nb * _KB
    nbq = q // _QB
    if kp != k:
        index_data = jnp.pad(index_data, ((0, kp - k), (0, 0)),
                             constant_values=_PAD_VAL)
    xt = index_data.T
    idx2 = indices.reshape(b, 1)

    return pl.pallas_call(
        functools.partial(_knn_body, nb=nb, nbq=nbq),
        grid=(nb, nbq),
        in_specs=[
            pl.BlockSpec((d, _KB), lambda i, j: (0, i)),
            pl.BlockSpec((_QB, d), lambda i, j: (j, 0)),
            pl.BlockSpec((b, 1), lambda i, j: (0, 0)),
        ],
        out_specs=pl.BlockSpec((b, _TOPK), lambda i, j: (0, 0)),
        out_shape=jax.ShapeDtypeStruct((b, _TOPK), jnp.float32),
        scratch_shapes=[pltpu.VMEM((q, _TOPK), jnp.float32),
                        pltpu.VMEM((_QB, _KB), jnp.int32)],
    )(xt, query_data, idx2)


# speedup vs baseline: 2.1495x; 1.3260x over previous
import functools

import jax
import jax.numpy as jnp
from jax.experimental import pallas as pl
from jax.experimental.pallas import tpu as pltpu

_TOPK = 8
_BETA = 1.0
_KB = 7168
_NC = 8
_CW = _KB // _NC
_QB = 128
_BB = 128
_COL_MASK = _CW - 1
_INT_MAX = jnp.iinfo(jnp.int32).max
_PAD_VAL = 1e17


def _knn_body(x_ref, q_ref, idx_ref, out_ref, top_ref, key_ref, *, nb, nbq):
    i = pl.program_id(0)
    j = pl.program_id(1)
    rows = pl.ds(j * _QB, _QB)

    @pl.when(i == 0)
    def _init():
        top_ref[rows, :] = jnp.full((_QB, _TOPK), jnp.inf, jnp.float32)

    q = q_ref[...]
    q2 = jnp.sum(q * q, axis=1, keepdims=True)
    col = jax.lax.broadcasted_iota(jnp.int32, (_QB, _CW), 1)

    rs0 = []
    for c in range(_NC):
        xc = x_ref[:, c * _CW:(c + 1) * _CW]
        gc = jax.lax.dot_general(q, xc, (((1,), (0,)), ((), ())),
                                 preferred_element_type=jnp.float32)
        x2c = jnp.sum(xc * xc, axis=0, keepdims=True)
        d2c = q2 + (x2c - 2.0 * gc)
        uc = jax.lax.bitcast_convert_type(d2c, jnp.int32)
        kc = (uc & ~_COL_MASK) | col
        key_ref[:, c * _CW:(c + 1) * _CW] = kc
        rs0.append(jnp.min(kc, axis=1, keepdims=True))

    top = top_ref[rows, :]
    neg_inf = jnp.full((_QB, 1), -jnp.inf, jnp.float32)

    def _val(m):
        return jnp.maximum(
            jax.lax.bitcast_convert_type(m & ~_COL_MASK, jnp.float32), 0.0)

    def _cond(carry):
        top, rs = carry
        m = functools.reduce(jnp.minimum, rs)
        return jnp.any(_val(m) < top[:, _TOPK - 1:])

    def _body(carry):
        top, rs = carry
        for r in rs:
            v = _val(r)
            shifted = jnp.concatenate([neg_inf, top[:, :_TOPK - 1]], axis=1)
            top = jnp.minimum(jnp.maximum(v, shifted), top)
        new_rs = []
        for c in range(_NC):
            kc = key_ref[:, c * _CW:(c + 1) * _CW]
            masked = jnp.where(kc <= rs[c], _INT_MAX, kc)
            new_rs.append(jnp.min(masked, axis=1, keepdims=True))
        return top, tuple(new_rs)

    top, _ = jax.lax.while_loop(_cond, _body, (top, tuple(rs0)))
    top_ref[rows, :] = top

    @pl.when((i == nb - 1) & (j == nbq - 1))
    def _final():
        w = jnp.exp(-_BETA * jnp.sqrt(top_ref[...] + 1e-12))
        nq = w.shape[0]
        nbb = idx_ref.shape[0] // _BB
        for bi in range(nbb):
            brows = pl.ds(bi * _BB, _BB)
            idx = idx_ref[brows, :]
            q_iota = jax.lax.broadcasted_iota(
                jnp.int32, (_BB, nq), 1)
            onehot = (q_iota == idx).astype(jnp.float32)
            out_ref[brows, :] = jax.lax.dot_general(
                onehot, w, (((1,), (0,)), ((), ())),
                preferred_element_type=jnp.float32)


@jax.jit
def kernel(index_data, query_data, indices):
    k, d = index_data.shape
    q, _ = query_data.shape
    b = indices.shape[0]
    nb = pl.cdiv(k, _KB)
    kp = nb * _KB
    nbq = q // _QB
    if kp != k:
        index_data = jnp.pad(index_data, ((0, kp - k), (0, 0)),
                             constant_values=_PAD_VAL)
    xt = index_data.T
    idx2 = indices.reshape(b, 1)

    return pl.pallas_call(
        functools.partial(_knn_body, nb=nb, nbq=nbq),
        grid=(nb, nbq),
        in_specs=[
            pl.BlockSpec((d, _KB), lambda i, j: (0, i)),
            pl.BlockSpec((_QB, d), lambda i, j: (j, 0)),
            pl.BlockSpec((b, 1), lambda i, j: (0, 0)),
        ],
        out_specs=pl.BlockSpec((b, _TOPK), lambda i, j: (0, 0)),
        out_shape=jax.ShapeDtypeStruct((b, _TOPK), jnp.float32),
        scratch_shapes=[pltpu.VMEM((q, _TOPK), jnp.float32),
                        pltpu.VMEM((_QB, _KB), jnp.int32)],
    )(xt, query_data, idx2)


# speedup vs baseline: 2.4394x; 1.1349x over previous
import functools

import jax
import jax.numpy as jnp
from jax.experimental import pallas as pl
from jax.experimental.pallas import tpu as pltpu

_TOPK = 8
_BETA = 1.0
_KB = 14336
_NC = 8
_CW = _KB // _NC
_QB = 128
_BB = 128
_COL_MASK = _CW - 1
_INT_MAX = jnp.iinfo(jnp.int32).max
_PAD_VAL = 1e17


def _knn_body(x_ref, q_ref, idx_ref, out_ref, top_ref, key_ref, *, nb, nbq):
    i = pl.program_id(0)
    j = pl.program_id(1)
    rows = pl.ds(j * _QB, _QB)

    @pl.when(i == 0)
    def _init():
        top_ref[rows, :] = jnp.full((_QB, _TOPK), jnp.inf, jnp.float32)

    q = q_ref[...]
    q2 = jnp.sum(q * q, axis=1, keepdims=True)
    col = jax.lax.broadcasted_iota(jnp.int32, (_QB, _CW), 1)

    rs0 = []
    for c in range(_NC):
        xc = x_ref[:, c * _CW:(c + 1) * _CW]
        gc = jax.lax.dot_general(q, xc, (((1,), (0,)), ((), ())),
                                 preferred_element_type=jnp.float32)
        x2c = jnp.sum(xc * xc, axis=0, keepdims=True)
        d2c = q2 + (x2c - 2.0 * gc)
        uc = jax.lax.bitcast_convert_type(d2c, jnp.int32)
        kc = (uc & ~_COL_MASK) | col
        key_ref[:, c * _CW:(c + 1) * _CW] = kc
        rs0.append(jnp.min(kc, axis=1, keepdims=True))

    top = top_ref[rows, :]
    neg_inf = jnp.full((_QB, 1), -jnp.inf, jnp.float32)

    def _val(m):
        return jnp.maximum(
            jax.lax.bitcast_convert_type(m & ~_COL_MASK, jnp.float32), 0.0)

    def _cond(carry):
        top, rs = carry
        m = functools.reduce(jnp.minimum, rs)
        return jnp.any(_val(m) < top[:, _TOPK - 1:])

    def _body(carry):
        top, rs = carry
        for r in rs:
            v = _val(r)
            shifted = jnp.concatenate([neg_inf, top[:, :_TOPK - 1]], axis=1)
            top = jnp.minimum(jnp.maximum(v, shifted), top)
        new_rs = []
        for c in range(_NC):
            kc = key_ref[:, c * _CW:(c + 1) * _CW]
            masked = jnp.where(kc <= rs[c], _INT_MAX, kc)
            new_rs.append(jnp.min(masked, axis=1, keepdims=True))
        return top, tuple(new_rs)

    top, _ = jax.lax.while_loop(_cond, _body, (top, tuple(rs0)))
    top_ref[rows, :] = top

    @pl.when((i == nb - 1) & (j == nbq - 1))
    def _final():
        w = jnp.exp(-_BETA * jnp.sqrt(top_ref[...] + 1e-12))
        nq = w.shape[0]
        nbb = idx_ref.shape[0] // _BB
        for bi in range(nbb):
            brows = pl.ds(bi * _BB, _BB)
            idx = idx_ref[brows, :]
            q_iota = jax.lax.broadcasted_iota(
                jnp.int32, (_BB, nq), 1)
            onehot = (q_iota == idx).astype(jnp.float32)
            out_ref[brows, :] = jax.lax.dot_general(
                onehot, w, (((1,), (0,)), ((), ())),
                preferred_element_type=jnp.float32)


@jax.jit
def kernel(index_data, query_data, indices):
    k, d = index_data.shape
    q, _ = query_data.shape
    b = indices.shape[0]
    nb = pl.cdiv(k, _KB)
    kp = nb * _KB
    nbq = q // _QB
    if kp != k:
        index_data = jnp.pad(index_data, ((0, kp - k), (0, 0)),
                             constant_values=_PAD_VAL)
    xt = index_data.T
    idx2 = indices.reshape(b, 1)

    return pl.pallas_call(
        functools.partial(_knn_body, nb=nb, nbq=nbq),
        grid=(nb, nbq),
        in_specs=[
            pl.BlockSpec((d, _KB), lambda i, j: (0, i)),
            pl.BlockSpec((_QB, d), lambda i, j: (j, 0)),
            pl.BlockSpec((b, 1), lambda i, j: (0, 0)),
        ],
        out_specs=pl.BlockSpec((b, _TOPK), lambda i, j: (0, 0)),
        out_shape=jax.ShapeDtypeStruct((b, _TOPK), jnp.float32),
        scratch_shapes=[pltpu.VMEM((q, _TOPK), jnp.float32),
                        pltpu.VMEM((_QB, _KB), jnp.int32)],
    )(xt, query_data, idx2)


# speedup vs baseline: 2.5405x; 1.0414x over previous
import functools

import jax
import jax.numpy as jnp
from jax.experimental import pallas as pl
from jax.experimental.pallas import tpu as pltpu

_TOPK = 8
_BETA = 1.0
_KB = 14336
_NC = 8
_CW = _KB // _NC
_QB = 128
_BB = 128
_COL_MASK = _CW - 1
_INT_MAX = jnp.iinfo(jnp.int32).max
_PAD_VAL = 1e17


def _knn_body(x_ref, q_ref, idx_ref, out_ref, top_ref, key_ref, *, nb, nbq):
    i = pl.program_id(0)
    j = pl.program_id(1)
    rows = pl.ds(j * _QB, _QB)

    @pl.when(i == 0)
    def _init():
        top_ref[rows, :] = jnp.full((_QB, _TOPK), jnp.inf, jnp.float32)

    q = q_ref[...]
    q2 = jnp.sum(q * q, axis=1, keepdims=True)
    col = jax.lax.broadcasted_iota(jnp.int32, (_QB, _CW), 1)

    rs1, rs2, rs3 = [], [], []
    for c in range(_NC):
        xc = x_ref[:, c * _CW:(c + 1) * _CW]
        gc = jax.lax.dot_general(q, xc, (((1,), (0,)), ((), ())),
                                 preferred_element_type=jnp.float32)
        x2c = jnp.sum(xc * xc, axis=0, keepdims=True)
        d2c = q2 + (x2c - 2.0 * gc)
        uc = jax.lax.bitcast_convert_type(d2c, jnp.int32)
        kc = (uc & ~_COL_MASK) | col
        key_ref[:, c * _CW:(c + 1) * _CW] = kc
        r1 = jnp.min(kc, axis=1, keepdims=True)
        kc = jnp.where(kc <= r1, _INT_MAX, kc)
        r2 = jnp.min(kc, axis=1, keepdims=True)
        kc = jnp.where(kc <= r2, _INT_MAX, kc)
        r3 = jnp.min(kc, axis=1, keepdims=True)
        rs1.append(r1)
        rs2.append(r2)
        rs3.append(r3)

    top = top_ref[rows, :]
    neg_inf = jnp.full((_QB, 1), -jnp.inf, jnp.float32)

    def _val(m):
        return jnp.maximum(
            jax.lax.bitcast_convert_type(m & ~_COL_MASK, jnp.float32), 0.0)

    def _insert(top, r):
        v = _val(r)
        shifted = jnp.concatenate([neg_inf, top[:, :_TOPK - 1]], axis=1)
        return jnp.minimum(jnp.maximum(v, shifted), top)

    for r in rs1 + rs2:
        top = _insert(top, r)

    def _cond(carry):
        top, rs = carry
        m = functools.reduce(jnp.minimum, rs)
        return jnp.any(_val(m) < top[:, _TOPK - 1:])

    def _body(carry):
        top, rs = carry
        for r in rs:
            top = _insert(top, r)
        new_rs = []
        for c in range(_NC):
            kc = key_ref[:, c * _CW:(c + 1) * _CW]
            masked = jnp.where(kc <= rs[c], _INT_MAX, kc)
            new_rs.append(jnp.min(masked, axis=1, keepdims=True))
        return top, tuple(new_rs)

    top, _ = jax.lax.while_loop(_cond, _body, (top, tuple(rs3)))
    top_ref[rows, :] = top

    @pl.when((i == nb - 1) & (j == nbq - 1))
    def _final():
        w = jnp.exp(-_BETA * jnp.sqrt(top_ref[...] + 1e-12))
        nq = w.shape[0]
        nbb = idx_ref.shape[0] // _BB
        for bi in range(nbb):
            brows = pl.ds(bi * _BB, _BB)
            idx = idx_ref[brows, :]
            q_iota = jax.lax.broadcasted_iota(
                jnp.int32, (_BB, nq), 1)
            onehot = (q_iota == idx).astype(jnp.float32)
            out_ref[brows, :] = jax.lax.dot_general(
                onehot, w, (((1,), (0,)), ((), ())),
                preferred_element_type=jnp.float32)


@jax.jit
def kernel(index_data, query_data, indices):
    k, d = index_data.shape
    q, _ = query_data.shape
    b = indices.shape[0]
    nb = pl.cdiv(k, _KB)
    kp = nb * _KB
    nbq = q // _QB
    if kp != k:
        index_data = jnp.pad(index_data, ((0, kp - k), (0, 0)),
                             constant_values=_PAD_VAL)
    xt = index_data.T
    idx2 = indices.reshape(b, 1)

    return pl.pallas_call(
        functools.partial(_knn_body, nb=nb, nbq=nbq),
        grid=(nb, nbq),
        in_specs=[
            pl.BlockSpec((d, _KB), lambda i, j: (0, i)),
            pl.BlockSpec((_QB, d), lambda i, j: (j, 0)),
            pl.BlockSpec((b, 1), lambda i, j: (0, 0)),
        ],
        out_specs=pl.BlockSpec((b, _TOPK), lambda i, j: (0, 0)),
        out_shape=jax.ShapeDtypeStruct((b, _TOPK), jnp.float32),
        scratch_shapes=[pltpu.VMEM((q, _TOPK), jnp.float32),
                        pltpu.VMEM((_QB, _KB), jnp.int32)],
    )(xt, query_data, idx2)
